# baseline (speedup 1.0000x reference)
import functools

import numpy as np
import jax
import jax.numpy as jnp
from jax import lax
from jax.experimental import pallas as pl
from jax.experimental.pallas import tpu as pltpu
from jax.experimental.pallas import tpu_sc as plsc

_N_DIM = 64
_N_BINS = 64
_R = 1.2
_BOUND = 50.0

_m = _N_BINS / 2
_x1L = _BOUND * (_R - 1.0) / (_R ** _m - 1.0)
_index = np.arange(0, _N_BINS + 1, dtype=np.float64).reshape(-1, 1) - _m
_xr = np.where(_index >= 0,
               (1.0 - np.power(_R, _index)) / (1.0 - _R),
               (1.0 - np.power(_R, np.abs(_index))) / (1.0 - _R))
_xr = np.where(_index >= 0, _x1L * _xr, -_x1L * _xr)
_xr = (_xr + _BOUND) / 2.0 / _BOUND
_X1L_S = float(_x1L / 2.0 / _BOUND)
_MESH = np.concatenate([np.array([[0.0]]), _xr[1:-1].reshape(-1, 1),
                        np.array([[1.0]])], 0).astype(np.float32)
_ELMT = (_MESH[1:] - _MESH[:-1]).reshape(-1).astype(np.float32)
_MESHC64 = (_MESH[:_N_BINS, 0].astype(np.float64) - 0.5).astype(np.float32)
_ELMT64X = np.broadcast_to(_ELMT.reshape(-1, 1), (_N_BINS, _N_DIM)).copy()
_MESHC64X = np.broadcast_to(_MESHC64.reshape(-1, 1), (_N_BINS, _N_DIM)).copy()
_ELMT0 = float(_ELMT[0])

_C_T = float((_R - 1.0) / _X1L_S)
_INV_LOG2_R = float(1.0 / np.log2(_R))
_T_CLAMP = 420.0
_LOG2_POLY = [-2.786813, 5.046876, -3.4924943, 1.5939014,
              -0.40486717, 0.04342891]


def _prep_body(p_ref, elmt_ref, mc_ref, a_ref, b_ref, c_ref):
    ep = jnp.exp(p_ref[...])
    elmt = elmt_ref[...]
    w = (elmt[:-1, :] + elmt[1:, :]) * 0.5
    denom = jnp.sum(ep * w, axis=0, keepdims=True)
    px = (jnp.float32(1.0 - _ELMT0) / denom) * ep
    ones = jnp.ones((1, _N_DIM), jnp.float32)
    v1 = jnp.concatenate([ones, px], 0)
    v2 = jnp.concatenate([px, ones], 0)
    cell = (v1 + v2) * 0.5 * elmt
    inc = cell
    for s in (1, 2, 4, 8, 16, 32):
        inc = inc + jnp.concatenate(
            [jnp.zeros((s, _N_DIM), jnp.float32), inc[:-s, :]], 0)
    f = jnp.concatenate([jnp.zeros((1, _N_DIM), jnp.float32),
                         inc[: _N_BINS - 1, :]], 0)
    mc = mc_ref[...]
    v1s = v1 * 100.0
    gs = (v2 - v1) * (50.0 / elmt)
    fs = f * 100.0 - 50.0
    a_ref[...] = fs - mc * v1s + mc * mc * gs
    b_ref[...] = v1s - 2.0 * mc * gs
    c_ref[...] = gs


_prep = pl.pallas_call(
    _prep_body,
    out_shape=[jax.ShapeDtypeStruct((_N_BINS, _N_DIM), jnp.float32)] * 3,
)


_NC = 2
_NS = 16
_NW = _NC * _NS
_LANES = 16
_N_ROWS = 262144
_ROWS_W = _N_ROWS // _NW
_CH_ROWS = 128
_N_CHUNKS = _ROWS_W // _CH_ROWS


def _sc_body(x_hbm, a_hbm, b_hbm, c_hbm, out_hbm,
             a_v, b_v, c_v, xin0, xin1, yout0, yout1,
             si0, si1, so0, so1):
    wid = lax.axis_index("s") * _NC + lax.axis_index("c")
    pltpu.sync_copy(a_hbm, a_v)
    pltpu.sync_copy(b_hbm, b_v)
    pltpu.sync_copy(c_hbm, c_v)

    iota = lax.iota(jnp.int32, _LANES)
    dvecs = [iota + (j * _LANES) for j in range(_N_DIM // _LANES)]

    def in_copy(c, buf, sem):
        return pltpu.make_async_copy(
            x_hbm.at[pl.ds(wid * _ROWS_W + c * _CH_ROWS, _CH_ROWS)], buf, sem)

    def out_copy(c, buf, sem):
        return pltpu.make_async_copy(
            buf, out_hbm.at[pl.ds(wid * _ROWS_W + c * _CH_ROWS, _CH_ROWS)], sem)

    def compute(xin, yout):
        @plsc.parallel_loop(0, _CH_ROWS, unroll=8)
        def row_body(r):
            for j in range(_N_DIM // _LANES):
                xv = xin[r, pl.ds(j * _LANES, _LANES)]
                u = xv * jnp.float32(1.0 / (2.0 * _BOUND))
                au = jnp.abs(u)
                t = au * jnp.float32(_C_T) + 1.0
                bits = lax.bitcast_convert_type(t, jnp.int32)
                e = lax.shift_right_logical(bits, 23) - 127
                mbits = jnp.bitwise_or(jnp.bitwise_and(bits, 0x7FFFFF),
                                       0x3F800000)
                mant = lax.bitcast_convert_type(mbits, jnp.float32)
                acc = jnp.float32(_LOG2_POLY[5])
                for cf in _LOG2_POLY[4::-1]:
                    acc = acc * mant + jnp.float32(cf)
                lg2 = acc + e.astype(jnp.float32)
                mfl = (lg2 * jnp.float32(_INV_LOG2_R)).astype(jnp.int32)
                k = jnp.where(u < 0.0, 31 - mfl, 32 + mfl)
                cover = mfl <= 31
                kc = jnp.clip(k, 0, _N_BINS - 1)
                fidx = kc * _N_DIM + dvecs[j]
                a = plsc.load_gather(a_v, [fidx])
                b = plsc.load_gather(b_v, [fidx])
                cq = plsc.load_gather(c_v, [fidx])
                yq = a + u * (b + u * cq)
                yv = jnp.where(cover, yq, u * jnp.float32(2.0 * _BOUND))
                yout[r, pl.ds(j * _LANES, _LANES)] = yv

    in_copy(0, xin0, si0).start()

    def pair_body(p, carry):
        c0 = p * 2
        in_copy(c0 + 1, xin1, si1).start()
        in_copy(c0, xin0, si0).wait()

        @pl.when(p > 0)
        def _():
            out_copy(c0, yout0, so0).wait()
        compute(xin0, yout0)
        out_copy(c0, yout0, so0).start()

        @pl.when(p < _N_CHUNKS // 2 - 1)
        def _():
            in_copy(c0 + 2, xin0, si0).start()

        @pl.when(p > 0)
        def _():
            out_copy(c0 + 1, yout1, so1).wait()
        in_copy(c0 + 1, xin1, si1).wait()
        compute(xin1, yout1)
        out_copy(c0 + 1, yout1, so1).start()
        return carry

    lax.fori_loop(0, _N_CHUNKS // 2, pair_body, 0)
    out_copy(_N_CHUNKS - 2, yout0, so0).wait()
    out_copy(_N_CHUNKS - 1, yout1, so1).wait()


@functools.cache
def _sc_main():
    return functools.partial(
        pl.kernel,
        out_type=jax.ShapeDtypeStruct((_N_ROWS, _N_DIM), jnp.float32),
        mesh=plsc.VectorSubcoreMesh(core_axis_name="c", subcore_axis_name="s",
                                    num_cores=_NC, num_subcores=_NS),
        compiler_params=pltpu.CompilerParams(needs_layout_passes=False),
        scratch_types=[
            pltpu.VMEM((_N_BINS * _N_DIM,), jnp.float32),
            pltpu.VMEM((_N_BINS * _N_DIM,), jnp.float32),
            pltpu.VMEM((_N_BINS * _N_DIM,), jnp.float32),
            pltpu.VMEM((_CH_ROWS, _N_DIM), jnp.float32),
            pltpu.VMEM((_CH_ROWS, _N_DIM), jnp.float32),
            pltpu.VMEM((_CH_ROWS, _N_DIM), jnp.float32),
            pltpu.VMEM((_CH_ROWS, _N_DIM), jnp.float32),
            pltpu.SemaphoreType.DMA,
            pltpu.SemaphoreType.DMA,
            pltpu.SemaphoreType.DMA,
            pltpu.SemaphoreType.DMA,
        ],
    )(_sc_body)


def kernel(inputs, p):
    a, b, c = _prep(p, jnp.asarray(_ELMT64X), jnp.asarray(_MESHC64X))
    return _sc_main()(inputs, a.reshape(-1), b.reshape(-1), c.reshape(-1))


# speedup vs baseline: 1.1413x; 1.1413x over previous
import functools

import numpy as np
import jax
import jax.numpy as jnp
from jax import lax
from jax.experimental import pallas as pl
from jax.experimental.pallas import tpu as pltpu
from jax.experimental.pallas import tpu_sc as plsc

_N_DIM = 64
_N_BINS = 64
_R = 1.2
_BOUND = 50.0

_m = _N_BINS / 2
_x1L = _BOUND * (_R - 1.0) / (_R ** _m - 1.0)
_index = np.arange(0, _N_BINS + 1, dtype=np.float64).reshape(-1, 1) - _m
_xr = np.where(_index >= 0,
               (1.0 - np.power(_R, _index)) / (1.0 - _R),
               (1.0 - np.power(_R, np.abs(_index))) / (1.0 - _R))
_xr = np.where(_index >= 0, _x1L * _xr, -_x1L * _xr)
_xr = (_xr + _BOUND) / 2.0 / _BOUND
_X1L_S = float(_x1L / 2.0 / _BOUND)
_MESH = np.concatenate([np.array([[0.0]]), _xr[1:-1].reshape(-1, 1),
                        np.array([[1.0]])], 0).astype(np.float32)
_ELMT = (_MESH[1:] - _MESH[:-1]).reshape(-1).astype(np.float32)
_MESHC64 = (_MESH[:_N_BINS, 0].astype(np.float64) - 0.5).astype(np.float32)
_ELMT64X = np.broadcast_to(_ELMT.reshape(-1, 1), (_N_BINS, _N_DIM)).copy()
_MESHC64X = np.broadcast_to(_MESHC64.reshape(-1, 1), (_N_BINS, _N_DIM)).copy()
_ELMT0 = float(_ELMT[0])

_C_T = float((_R - 1.0) / _X1L_S)
_INV_LOG2_R = float(1.0 / np.log2(_R))
_T_CLAMP = 420.0
_LOG2_POLY = [-2.786813, 5.046876, -3.4924943, 1.5939014,
              -0.40486717, 0.04342891]


def _prep_body(p_ref, elmt_ref, mc_ref, a_ref, b_ref, c_ref):
    ep = jnp.exp(p_ref[...])
    elmt = elmt_ref[...]
    w = (elmt[:-1, :] + elmt[1:, :]) * 0.5
    denom = jnp.sum(ep * w, axis=0, keepdims=True)
    px = (jnp.float32(1.0 - _ELMT0) / denom) * ep
    ones = jnp.ones((1, _N_DIM), jnp.float32)
    v1 = jnp.concatenate([ones, px], 0)
    v2 = jnp.concatenate([px, ones], 0)
    cell = (v1 + v2) * 0.5 * elmt
    inc = cell
    for s in (1, 2, 4, 8, 16, 32):
        inc = inc + jnp.concatenate(
            [jnp.zeros((s, _N_DIM), jnp.float32), inc[:-s, :]], 0)
    f = jnp.concatenate([jnp.zeros((1, _N_DIM), jnp.float32),
                         inc[: _N_BINS - 1, :]], 0)
    mc = mc_ref[...]
    v1s = v1 * 100.0
    gs = (v2 - v1) * (50.0 / elmt)
    fs = f * 100.0 - 50.0
    a_ref[...] = fs - mc * v1s + mc * mc * gs
    b_ref[...] = v1s - 2.0 * mc * gs
    c_ref[...] = gs


_prep = pl.pallas_call(
    _prep_body,
    out_shape=[jax.ShapeDtypeStruct((_N_BINS, _N_DIM), jnp.float32)] * 3,
)


_NC = 2
_NS = 16
_NW = _NC * _NS
_LANES = 16
_N_ROWS = 262144
_ROWS_W = _N_ROWS // _NW
_CH_ROWS = 128
_N_CHUNKS = _ROWS_W // _CH_ROWS


def _sc_body(x_hbm, a_hbm, b_hbm, c_hbm, out_hbm,
             a_v, b_v, c_v, xin0, xin1, yout0, yout1,
             si0, si1, so0, so1):
    wid = lax.axis_index("s") * _NC + lax.axis_index("c")
    pltpu.sync_copy(a_hbm, a_v)
    pltpu.sync_copy(b_hbm, b_v)
    pltpu.sync_copy(c_hbm, c_v)

    iota = lax.iota(jnp.int32, _LANES)
    dvecs = [iota + (j * _LANES) for j in range(_N_DIM // _LANES)]

    def in_copy(c, buf, sem):
        return pltpu.make_async_copy(
            x_hbm.at[pl.ds(wid * _ROWS_W + c * _CH_ROWS, _CH_ROWS)], buf, sem)

    def out_copy(c, buf, sem):
        return pltpu.make_async_copy(
            buf, out_hbm.at[pl.ds(wid * _ROWS_W + c * _CH_ROWS, _CH_ROWS)], sem)

    def compute(xin, yout):
        @plsc.parallel_loop(0, _CH_ROWS, unroll=4)
        def row_body(r):
            for j in range(_N_DIM // _LANES):
                xv = xin[r, pl.ds(j * _LANES, _LANES)]
                u = xv * jnp.float32(1.0 / (2.0 * _BOUND))
                au = jnp.abs(u)
                t = au * jnp.float32(_C_T) + 1.0
                bits = lax.bitcast_convert_type(t, jnp.int32)
                e = lax.shift_right_logical(bits, 23) - 127
                mbits = jnp.bitwise_or(jnp.bitwise_and(bits, 0x7FFFFF),
                                       0x3F800000)
                mant = lax.bitcast_convert_type(mbits, jnp.float32)
                acc = jnp.float32(_LOG2_POLY[5])
                for cf in _LOG2_POLY[4::-1]:
                    acc = acc * mant + jnp.float32(cf)
                lg2 = acc + e.astype(jnp.float32)
                mfl = (lg2 * jnp.float32(_INV_LOG2_R)).astype(jnp.int32)
                k = jnp.where(u < 0.0, 31 - mfl, 32 + mfl)
                cover = mfl <= 31
                kc = jnp.clip(k, 0, _N_BINS - 1)
                fidx = kc * _N_DIM + dvecs[j]
                a = plsc.load_gather(a_v, [fidx])
                b = plsc.load_gather(b_v, [fidx])
                cq = plsc.load_gather(c_v, [fidx])
                yq = a + u * (b + u * cq)
                yv = jnp.where(cover, yq, u * jnp.float32(2.0 * _BOUND))
                yout[r, pl.ds(j * _LANES, _LANES)] = yv

    in_copy(0, xin0, si0).start()

    def pair_body(p, carry):
        c0 = p * 2
        in_copy(c0 + 1, xin1, si1).start()
        in_copy(c0, xin0, si0).wait()

        @pl.when(p > 0)
        def _():
            out_copy(c0, yout0, so0).wait()
        compute(xin0, yout0)
        out_copy(c0, yout0, so0).start()

        @pl.when(p < _N_CHUNKS // 2 - 1)
        def _():
            in_copy(c0 + 2, xin0, si0).start()

        @pl.when(p > 0)
        def _():
            out_copy(c0 + 1, yout1, so1).wait()
        in_copy(c0 + 1, xin1, si1).wait()
        compute(xin1, yout1)
        out_copy(c0 + 1, yout1, so1).start()
        return carry

    lax.fori_loop(0, _N_CHUNKS // 2, pair_body, 0)
    out_copy(_N_CHUNKS - 2, yout0, so0).wait()
    out_copy(_N_CHUNKS - 1, yout1, so1).wait()


@functools.cache
def _sc_main():
    return functools.partial(
        pl.kernel,
        out_type=jax.ShapeDtypeStruct((_N_ROWS, _N_DIM), jnp.float32),
        mesh=plsc.VectorSubcoreMesh(core_axis_name="c", subcore_axis_name="s",
                                    num_cores=_NC, num_subcores=_NS),
        compiler_params=pltpu.CompilerParams(needs_layout_passes=False),
        scratch_types=[
            pltpu.VMEM((_N_BINS * _N_DIM,), jnp.float32),
            pltpu.VMEM((_N_BINS * _N_DIM,), jnp.float32),
            pltpu.VMEM((_N_BINS * _N_DIM,), jnp.float32),
            pltpu.VMEM((_CH_ROWS, _N_DIM), jnp.float32),
            pltpu.VMEM((_CH_ROWS, _N_DIM), jnp.float32),
            pltpu.VMEM((_CH_ROWS, _N_DIM), jnp.float32),
            pltpu.VMEM((_CH_ROWS, _N_DIM), jnp.float32),
            pltpu.SemaphoreType.DMA,
            pltpu.SemaphoreType.DMA,
            pltpu.SemaphoreType.DMA,
            pltpu.SemaphoreType.DMA,
        ],
    )(_sc_body)


def kernel(inputs, p):
    a, b, c = _prep(p, jnp.asarray(_ELMT64X), jnp.asarray(_MESHC64X))
    return _sc_main()(inputs, a.reshape(-1), b.reshape(-1), c.reshape(-1))
